# baseline (speedup 1.0000x reference)
import functools

import jax
import jax.numpy as jnp
from jax.experimental import pallas as pl
from jax.experimental.pallas import tpu as pltpu

_BIMG = 16


def _swish(z):
    return z * (0.5 + 0.5 * jnp.tanh(0.5 * z))


def _mxu_stats(yb, ones_row):
    c = yb.shape[1]
    s = jax.lax.dot_general(ones_row, yb, (((1,), (0,)), ((), ())),
                            preferred_element_type=jnp.float32)
    g = jax.lax.dot_general(yb, yb, (((0,), (0,)), ((), ())),
                            preferred_element_type=jnp.float32)
    eye = (jax.lax.broadcasted_iota(jnp.int32, (c, c), 0) ==
           jax.lax.broadcasted_iota(jnp.int32, (c, c), 1))
    sq = jnp.sum(jnp.where(eye, g, 0.0), axis=0, keepdims=True)
    return s, sq


def _s1_kernel(x_ref, w_ref, b_ref, y_ref, st_ref):
    p_io = jax.lax.broadcasted_iota(jnp.int32, (576, 256), 0)
    l_io = jax.lax.broadcasted_iota(jnp.int32, (576, 256), 1)
    mask = ((8 * p_io + l_io // 32) % 72) < 64
    c256 = jax.lax.broadcasted_iota(jnp.int32, (256, 256), 0)
    eye256 = c256 == jax.lax.broadcasted_iota(jnp.int32, (256, 256), 1)
    fold = jnp.where(
        (jax.lax.broadcasted_iota(jnp.int32, (256, 32), 0) % 32) ==
        jax.lax.broadcasted_iota(jnp.int32, (256, 32), 1), 1.0, 0.0)
    ones_row = jnp.ones((1, 576), jnp.bfloat16)
    for b in range(x_ref.shape[0]):
        a = x_ref[b]
        t0 = a[0:576, :]
        t1 = jnp.concatenate([a[0:576, 12:96], a[1:577, 0:12]], axis=1)
        t2 = a[9:585, :]
        t3 = jnp.concatenate([a[9:585, 12:96], a[10:586, 0:12]], axis=1)
        patches = jnp.concatenate([t0, t1, t2, t3], axis=1)
        acc = jnp.dot(patches, w_ref[...],
                      preferred_element_type=jnp.float32) + b_ref[...]
        acc = jnp.where(mask, acc, 0.0)
        yb = acc.astype(jnp.bfloat16)
        y_ref[b] = yb
        s = jax.lax.dot_general(ones_row, yb, (((1,), (0,)), ((), ())),
                                preferred_element_type=jnp.float32)
        g = jax.lax.dot_general(yb, yb, (((0,), (0,)), ((), ())),
                                preferred_element_type=jnp.float32)
        sq = jnp.sum(jnp.where(eye256, g, 0.0), axis=0, keepdims=True)
        st_ref[b] = jax.lax.dot_general(
            jnp.concatenate([s, sq], axis=0), fold,
            (((1,), (0,)), ((), ())), preferred_element_type=jnp.float32)


def _conv_kernel(x_ref, w_ref, b_ref, scale_ref, shift_ref,
                 y_ref, st_ref, pad_ref, *, H, W_in, Cin, Cout):
    Ho = H // 2
    ones_row = jnp.ones((1, Ho * Ho), jnp.bfloat16)
    pad_ref[0:1, :, :] = jnp.zeros_like(pad_ref[0:1, :, :])
    pad_ref[H + 1:H + 2, :, :] = jnp.zeros_like(pad_ref[H + 1:H + 2, :, :])
    pad_ref[:, 0:1, :] = jnp.zeros_like(pad_ref[:, 0:1, :])
    pad_ref[:, H + 1:, :] = jnp.zeros_like(pad_ref[:, H + 1:, :])
    for b in range(x_ref.shape[0]):
        x = x_ref[b].astype(jnp.float32)
        z = x * scale_ref[...] + shift_ref[...]
        a = _swish(z).reshape(H, W_in, Cin)
        pad_ref[1:H + 1, 1:H + 1, :] = a[:, 0:H, :]
        taps = []
        for kh in range(3):
            for kw in range(3):
                win = pad_ref[pl.ds(kh, Ho, stride=2),
                              pl.ds(kw, Ho, stride=2), :]
                taps.append(win.reshape(Ho * Ho, Cin).astype(jnp.bfloat16))
        patches = jnp.concatenate(taps, axis=1)
        acc = jnp.dot(patches, w_ref[...],
                      preferred_element_type=jnp.float32) + b_ref[...]
        yb = acc.astype(jnp.bfloat16)
        y_ref[b] = yb
        s, sq = _mxu_stats(yb, ones_row)
        st_ref[b] = jnp.concatenate([s, sq], axis=0)


def _head_kernel(x_ref, scale_ref, shift_ref, w_ref, b_ref, o_ref, *, S):
    denom = float((S + 2) * (S + 2))
    for b in range(x_ref.shape[0]):
        x = x_ref[b].astype(jnp.float32)
        z = x * scale_ref[...] + shift_ref[...]
        a = _swish(z)
        s = jnp.sum(a, axis=0, keepdims=True)
        logit = (jnp.sum(s * w_ref[...], axis=1, keepdims=True) / denom
                 + b_ref[...])
        o_ref[b] = 0.5 + 0.5 * jnp.tanh(0.5 * logit)


def _stage1(x_nchw, w2, b2):
    N = x_nchw.shape[0]
    t = x_nchw.reshape(N, 3, 64, 2, 64, 2)
    t = jnp.transpose(t, (0, 2, 4, 3, 5, 1)).reshape(N, 64, 64, 12)
    t = jnp.pad(t, ((0, 0), (0, 0), (1, 7), (0, 0)))
    t = t.reshape(N, 64 * 72, 12)
    t = jnp.pad(t, ((0, 0), (72, 56), (0, 0))).astype(jnp.bfloat16)
    t = t.reshape(N, 592, 96)

    kmap = {(0, 1): 0, (1, 0): 1, (1, 1): 2}
    rows = []
    for da, db in ((0, 0), (0, 1), (1, 0), (1, 1)):
        for dr in (0, 1):
            for dc in (0, 1):
                kh = kmap.get((da, dr))
                kw = kmap.get((db, dc))
                if kh is None or kw is None:
                    rows.append(jnp.zeros((3, 32), jnp.float32))
                else:
                    rows.append(w2[:, :, kh, kw].T)
    w48 = jnp.concatenate(rows, axis=0)
    blocks = []
    for tap in range(4):
        for g in range(8):
            blocks.append(jnp.pad(w48[12 * tap:12 * (tap + 1), :],
                                  ((0, 0), (32 * g, 256 - 32 * g - 32))))
    w_pack = jnp.concatenate(blocks, axis=0).astype(jnp.bfloat16)
    b_pack = jnp.tile(b2.reshape(1, 32), (1, 8)).astype(jnp.float32)

    B = _BIMG
    y, st = pl.pallas_call(
        _s1_kernel,
        out_shape=(
            jax.ShapeDtypeStruct((N, 576, 256), jnp.bfloat16),
            jax.ShapeDtypeStruct((N, 2, 32), jnp.float32),
        ),
        grid=(N // B,),
        in_specs=[
            pl.BlockSpec((B, 592, 96), lambda n: (n, 0, 0)),
            pl.BlockSpec((384, 256), lambda n: (0, 0)),
            pl.BlockSpec((1, 256), lambda n: (0, 0)),
        ],
        out_specs=(
            pl.BlockSpec((B, 576, 256), lambda n: (n, 0, 0)),
            pl.BlockSpec((B, 2, 32), lambda n: (n, 0, 0)),
        ),
        compiler_params=pltpu.CompilerParams(
            dimension_semantics=("parallel",)),
    )(t, w_pack, b_pack)
    return y.reshape(N, 4608, 32), st


def _conv_stage(x, w, b, scale, shift, *, H, W_in):
    N = x.shape[0]
    Cin, Cout = w.shape[1], w.shape[0]
    Ho = H // 2
    Wpad = ((H + 2 + 7) // 8) * 8
    w_mat = jnp.transpose(w, (2, 3, 1, 0)).reshape(9 * Cin, Cout)
    w_mat = w_mat.astype(jnp.bfloat16)
    kern = functools.partial(_conv_kernel, H=H, W_in=W_in, Cin=Cin, Cout=Cout)
    B = _BIMG
    y, st = pl.pallas_call(
        kern,
        out_shape=(
            jax.ShapeDtypeStruct((N, Ho * Ho, Cout), jnp.bfloat16),
            jax.ShapeDtypeStruct((N, 2, Cout), jnp.float32),
        ),
        grid=(N // B,),
        in_specs=[
            pl.BlockSpec((B, H * W_in, Cin), lambda n: (n, 0, 0)),
            pl.BlockSpec((9 * Cin, Cout), lambda n: (0, 0)),
            pl.BlockSpec((1, Cout), lambda n: (0, 0)),
            pl.BlockSpec((1, Cin), lambda n: (0, 0)),
            pl.BlockSpec((1, Cin), lambda n: (0, 0)),
        ],
        out_specs=(
            pl.BlockSpec((B, Ho * Ho, Cout), lambda n: (n, 0, 0)),
            pl.BlockSpec((B, 2, Cout), lambda n: (n, 0, 0)),
        ),
        scratch_shapes=[pltpu.VMEM((H + 2, Wpad, Cin), jnp.float32)],
        compiler_params=pltpu.CompilerParams(
            dimension_semantics=("parallel",)),
    )(x, w_mat, b.reshape(1, Cout), scale.reshape(1, Cin),
      shift.reshape(1, Cin))
    return y, st


def _head(y3, scale, shift, w9, b9, *, S):
    N, SS, C = y3.shape
    kern = functools.partial(_head_kernel, S=S)
    B = _BIMG
    out = pl.pallas_call(
        kern,
        out_shape=jax.ShapeDtypeStruct((N, 1, 1), jnp.float32),
        grid=(N // B,),
        in_specs=[
            pl.BlockSpec((B, SS, C), lambda n: (n, 0, 0)),
            pl.BlockSpec((1, C), lambda n: (0, 0)),
            pl.BlockSpec((1, C), lambda n: (0, 0)),
            pl.BlockSpec((1, C), lambda n: (0, 0)),
            pl.BlockSpec((1, 1), lambda n: (0, 0)),
        ],
        out_specs=pl.BlockSpec((B, 1, 1), lambda n: (n, 0, 0)),
        compiler_params=pltpu.CompilerParams(
            dimension_semantics=("parallel",)),
    )(y3, scale.reshape(1, C), shift.reshape(1, C),
      w9.reshape(1, C), b9.reshape(1, 1))
    return out.reshape(N, 1)


def _fold(st, count, gamma, beta, eps=1e-5):
    mean = jnp.sum(st[:, 0, :], axis=0) / count
    var = jnp.sum(st[:, 1, :], axis=0) / count - mean * mean
    scale = gamma * jax.lax.rsqrt(jnp.maximum(var, 0.0) + eps)
    return scale, beta - mean * scale


@jax.jit
def kernel(x_nchw, w2, b2, g2, be2, w4, b4, g4, be4, w6, b6, g6, be6, w9, b9):
    N = x_nchw.shape[0]
    y1, st1 = _stage1(x_nchw, w2, b2)
    sc1, sh1 = _fold(st1, N * 64 * 64, g2, be2)
    y2, st2 = _conv_stage(y1, w4, b4, sc1, sh1, H=64, W_in=72)
    sc2, sh2 = _fold(st2, N * 32 * 32, g4, be4)
    y3, st3 = _conv_stage(y2, w6, b6, sc2, sh2, H=32, W_in=32)
    sc3, sh3 = _fold(st3, N * 16 * 16, g6, be6)
    return _head(y3, sc3, sh3, w9, b9, S=16)


# speedup vs baseline: 1.0264x; 1.0264x over previous
import functools

import jax
import jax.numpy as jnp
from jax.experimental import pallas as pl
from jax.experimental.pallas import tpu as pltpu

_BIMG = 8


def _swish(z):
    return z * (0.5 + 0.5 * jnp.tanh(0.5 * z))


def _mxu_stats(yb, ones_row):
    c = yb.shape[1]
    s = jax.lax.dot_general(ones_row, yb, (((1,), (0,)), ((), ())),
                            preferred_element_type=jnp.float32)
    g = jax.lax.dot_general(yb, yb, (((0,), (0,)), ((), ())),
                            preferred_element_type=jnp.float32)
    eye = (jax.lax.broadcasted_iota(jnp.int32, (c, c), 0) ==
           jax.lax.broadcasted_iota(jnp.int32, (c, c), 1))
    sq = jnp.sum(jnp.where(eye, g, 0.0), axis=0, keepdims=True)
    return s, sq


def _s1_kernel(x_ref, w_ref, b_ref, y_ref, st_ref):
    mask = (jax.lax.broadcasted_iota(jnp.int32, (4608, 1), 0) % 72) < 64
    ones_row = jnp.ones((1, 4608), jnp.bfloat16)
    for b in range(x_ref.shape[0]):
        x = x_ref[b]
        taps = [x[0:4608, :], x[1:4609, :], x[72:4680, :], x[73:4681, :]]
        patches = jnp.concatenate(taps, axis=1)
        acc = jnp.dot(patches, w_ref[...],
                      preferred_element_type=jnp.float32) + b_ref[...]
        acc = jnp.where(mask, acc, 0.0)
        yb = acc.astype(jnp.bfloat16)
        y_ref[b] = yb
        s, sq = _mxu_stats(yb, ones_row)
        st_ref[b] = jnp.concatenate([s, sq], axis=0)


def _conv_kernel(x_ref, w_ref, b_ref, scale_ref, shift_ref,
                 y_ref, st_ref, pad_ref, *, H, W_in, Cin, Cout):
    Ho = H // 2
    ones_row = jnp.ones((1, Ho * Ho), jnp.bfloat16)
    pad_ref[0:1, :, :] = jnp.zeros_like(pad_ref[0:1, :, :])
    pad_ref[H + 1:H + 2, :, :] = jnp.zeros_like(pad_ref[H + 1:H + 2, :, :])
    pad_ref[:, 0:1, :] = jnp.zeros_like(pad_ref[:, 0:1, :])
    pad_ref[:, H + 1:, :] = jnp.zeros_like(pad_ref[:, H + 1:, :])
    for b in range(x_ref.shape[0]):
        x = x_ref[b].astype(jnp.float32)
        z = x * scale_ref[...] + shift_ref[...]
        a = _swish(z).reshape(H, W_in, Cin)
        pad_ref[1:H + 1, 1:H + 1, :] = a[:, 0:H, :]
        taps = []
        for kh in range(3):
            for kw in range(3):
                win = pad_ref[pl.ds(kh, Ho, stride=2),
                              pl.ds(kw, Ho, stride=2), :]
                taps.append(win.reshape(Ho * Ho, Cin).astype(jnp.bfloat16))
        patches = jnp.concatenate(taps, axis=1)
        acc = jnp.dot(patches, w_ref[...],
                      preferred_element_type=jnp.float32) + b_ref[...]
        yb = acc.astype(jnp.bfloat16)
        y_ref[b] = yb
        s, sq = _mxu_stats(yb, ones_row)
        st_ref[b] = jnp.concatenate([s, sq], axis=0)


def _head_kernel(x_ref, scale_ref, shift_ref, w_ref, b_ref, o_ref, *, S):
    denom = float((S + 2) * (S + 2))
    for b in range(x_ref.shape[0]):
        x = x_ref[b].astype(jnp.float32)
        z = x * scale_ref[...] + shift_ref[...]
        a = _swish(z)
        s = jnp.sum(a, axis=0, keepdims=True)
        logit = (jnp.sum(s * w_ref[...], axis=1, keepdims=True) / denom
                 + b_ref[...])
        o_ref[b] = 0.5 + 0.5 * jnp.tanh(0.5 * logit)


def _stage1(x_nchw, w2, b2):
    N = x_nchw.shape[0]
    t = x_nchw.reshape(N, 3, 64, 2, 64, 2)
    t = jnp.transpose(t, (0, 2, 4, 3, 5, 1)).reshape(N, 64, 64, 12)
    t = jnp.pad(t, ((0, 0), (0, 0), (1, 7), (0, 0)))
    t = t.reshape(N, 64 * 72, 12)
    t = jnp.pad(t, ((0, 0), (72, 8), (0, 0))).astype(jnp.bfloat16)

    kmap = {(0, 1): 0, (1, 0): 1, (1, 1): 2}
    rows = []
    for da, db in ((0, 0), (0, 1), (1, 0), (1, 1)):
        for dr in (0, 1):
            for dc in (0, 1):
                kh = kmap.get((da, dr))
                kw = kmap.get((db, dc))
                if kh is None or kw is None:
                    rows.append(jnp.zeros((3, 32), jnp.float32))
                else:
                    rows.append(w2[:, :, kh, kw].T)
    w_mat = jnp.concatenate(rows, axis=0).astype(jnp.bfloat16)

    B = _BIMG
    y, st = pl.pallas_call(
        _s1_kernel,
        out_shape=(
            jax.ShapeDtypeStruct((N, 4608, 32), jnp.bfloat16),
            jax.ShapeDtypeStruct((N, 2, 32), jnp.float32),
        ),
        grid=(N // B,),
        in_specs=[
            pl.BlockSpec((B, 4688, 12), lambda n: (n, 0, 0)),
            pl.BlockSpec((48, 32), lambda n: (0, 0)),
            pl.BlockSpec((1, 32), lambda n: (0, 0)),
        ],
        out_specs=(
            pl.BlockSpec((B, 4608, 32), lambda n: (n, 0, 0)),
            pl.BlockSpec((B, 2, 32), lambda n: (n, 0, 0)),
        ),
        compiler_params=pltpu.CompilerParams(
            dimension_semantics=("parallel",)),
    )(t, w_mat, b2.reshape(1, 32).astype(jnp.float32))
    return y, st


def _conv_stage(x, w, b, scale, shift, *, H, W_in):
    N = x.shape[0]
    Cin, Cout = w.shape[1], w.shape[0]
    Ho = H // 2
    Wpad = ((H + 2 + 7) // 8) * 8
    w_mat = jnp.transpose(w, (2, 3, 1, 0)).reshape(9 * Cin, Cout)
    w_mat = w_mat.astype(jnp.bfloat16)
    kern = functools.partial(_conv_kernel, H=H, W_in=W_in, Cin=Cin, Cout=Cout)
    B = _BIMG
    y, st = pl.pallas_call(
        kern,
        out_shape=(
            jax.ShapeDtypeStruct((N, Ho * Ho, Cout), jnp.bfloat16),
            jax.ShapeDtypeStruct((N, 2, Cout), jnp.float32),
        ),
        grid=(N // B,),
        in_specs=[
            pl.BlockSpec((B, H * W_in, Cin), lambda n: (n, 0, 0)),
            pl.BlockSpec((9 * Cin, Cout), lambda n: (0, 0)),
            pl.BlockSpec((1, Cout), lambda n: (0, 0)),
            pl.BlockSpec((1, Cin), lambda n: (0, 0)),
            pl.BlockSpec((1, Cin), lambda n: (0, 0)),
        ],
        out_specs=(
            pl.BlockSpec((B, Ho * Ho, Cout), lambda n: (n, 0, 0)),
            pl.BlockSpec((B, 2, Cout), lambda n: (n, 0, 0)),
        ),
        scratch_shapes=[pltpu.VMEM((H + 2, Wpad, Cin), jnp.float32)],
        compiler_params=pltpu.CompilerParams(
            dimension_semantics=("parallel",)),
    )(x, w_mat, b.reshape(1, Cout), scale.reshape(1, Cin),
      shift.reshape(1, Cin))
    return y, st


def _head(y3, scale, shift, w9, b9, *, S):
    N, SS, C = y3.shape
    kern = functools.partial(_head_kernel, S=S)
    B = _BIMG
    out = pl.pallas_call(
        kern,
        out_shape=jax.ShapeDtypeStruct((N, 1, 1), jnp.float32),
        grid=(N // B,),
        in_specs=[
            pl.BlockSpec((B, SS, C), lambda n: (n, 0, 0)),
            pl.BlockSpec((1, C), lambda n: (0, 0)),
            pl.BlockSpec((1, C), lambda n: (0, 0)),
            pl.BlockSpec((1, C), lambda n: (0, 0)),
            pl.BlockSpec((1, 1), lambda n: (0, 0)),
        ],
        out_specs=pl.BlockSpec((B, 1, 1), lambda n: (n, 0, 0)),
        compiler_params=pltpu.CompilerParams(
            dimension_semantics=("parallel",)),
    )(y3, scale.reshape(1, C), shift.reshape(1, C),
      w9.reshape(1, C), b9.reshape(1, 1))
    return out.reshape(N, 1)


def _fold(st, count, gamma, beta, eps=1e-5):
    mean = jnp.sum(st[:, 0, :], axis=0) / count
    var = jnp.sum(st[:, 1, :], axis=0) / count - mean * mean
    scale = gamma * jax.lax.rsqrt(jnp.maximum(var, 0.0) + eps)
    return scale, beta - mean * scale


@jax.jit
def kernel(x_nchw, w2, b2, g2, be2, w4, b4, g4, be4, w6, b6, g6, be6, w9, b9):
    N = x_nchw.shape[0]
    y1, st1 = _stage1(x_nchw, w2, b2)
    sc1, sh1 = _fold(st1, N * 64 * 64, g2, be2)
    y2, st2 = _conv_stage(y1, w4, b4, sc1, sh1, H=64, W_in=72)
    sc2, sh2 = _fold(st2, N * 32 * 32, g4, be4)
    y3, st3 = _conv_stage(y2, w6, b6, sc2, sh2, H=32, W_in=32)
    sc3, sh3 = _fold(st3, N * 16 * 16, g6, be6)
    return _head(y3, sc3, sh3, w9, b9, S=16)
